# baseline (speedup 1.0000x reference)
import jax
import jax.numpy as jnp
from jax.experimental import pallas as pl
from jax.experimental.pallas import tpu as pltpu

B_ORT, B_REG, B_CLS = 0.2, 2.0, 1.0
_B = 4
_P = 40000
_L = 5000
_NI = _P // _L
_CLS_TOTAL = float(_B * 2 * _P)


def _loss_kernel(x_ref, t_ref, rg_ref, rt_ref, out_ref, acc_ref):
    b = pl.program_id(0)
    i = pl.program_id(1)

    @pl.when(jnp.logical_and(b == 0, i == 0))
    def _init():
        for k in range(4):
            acc_ref[k] = 0.0

    x = x_ref[0, :, pl.ds(i, 1), :].reshape(2, _L)
    t = jnp.transpose(t_ref[0, 0])
    p = jax.nn.sigmoid(x)
    pt = jnp.where(t == 1.0, p, 1.0 - p)
    at = jnp.where(t == 1.0, 1000.0, 1.0)
    q = 1.0 - pt
    w = at * q * q
    bce = jnp.maximum(x, 0.0) - x * t + jnp.log1p(jnp.exp(-jnp.abs(x)))
    cls_sum = jnp.sum(w * bce)

    rt = jnp.transpose(rt_ref[0, 0])
    sl1_sum = 0.0
    ce_sum = 0.0
    npos = 0.0
    rows7 = jax.lax.broadcasted_iota(jnp.int32, (7, _L), 0)
    for a in range(2):
        mask = (rt[9 * a:9 * a + 1] == 1.0).astype(jnp.float32)
        npos += jnp.sum(mask)
        s = rg_ref[0, 9 * a:9 * a + 7, pl.ds(i, 1), :].reshape(7, _L)
        if a == 0:
            s = jnp.where(rows7 == 6, jnp.tanh(s), s)
        d = s - rt[9 * a + 1:9 * a + 8]
        ad = jnp.abs(d)
        sl1 = jnp.where(ad < 1.0, 0.5 * d * d, ad - 0.5)
        sl1_sum += jnp.sum(sl1 * mask)
        z = rg_ref[0, 9 * a + 7:9 * a + 9, pl.ds(i, 1), :].reshape(2, _L)
        z0 = z[0:1]
        z1 = z[1:2]
        tc = rt[9 * a + 8:9 * a + 9]
        diff = jnp.where(tc == 1.0, z0 - z1, z1 - z0)
        ce = jnp.maximum(diff, 0.0) + jnp.log1p(jnp.exp(-jnp.abs(diff)))
        ce_sum += jnp.sum(ce * mask)

    acc_ref[0] += cls_sum
    acc_ref[1] += sl1_sum
    acc_ref[2] += ce_sum
    acc_ref[3] += npos

    @pl.when(jnp.logical_and(b == _B - 1, i == _NI - 1))
    def _final():
        n_pos = acc_ref[3]
        cls_loss = acc_ref[0] / _CLS_TOTAL
        reg_loss = acc_ref[1] / (n_pos * 7.0)
        ort_loss = acc_ref[2] / n_pos
        loss = B_CLS * cls_loss + B_ORT * ort_loss + B_REG * reg_loss
        out_ref[...] = jnp.full((1, 1), loss, dtype=jnp.float32)


def kernel(cls_tensor, reg_tensor, cls_targets, reg_targets):
    x = cls_tensor.reshape(_B, 2, _NI, _L)
    t = cls_targets.reshape(_B, _NI, _L, 2)
    rg = reg_tensor.reshape(_B, 18, _NI, _L)
    rt = reg_targets.reshape(_B, _NI, _L, 18)

    out = pl.pallas_call(
        _loss_kernel,
        grid=(_B, _NI),
        in_specs=[
            pl.BlockSpec((1, 2, _NI, _L), lambda b, i: (b, 0, 0, 0)),
            pl.BlockSpec((1, 1, _L, 2), lambda b, i: (b, i, 0, 0)),
            pl.BlockSpec((1, 18, _NI, _L), lambda b, i: (b, 0, 0, 0)),
            pl.BlockSpec((1, 1, _L, 18), lambda b, i: (b, i, 0, 0)),
        ],
        out_specs=pl.BlockSpec((1, 1), lambda b, i: (0, 0)),
        out_shape=jax.ShapeDtypeStruct((1, 1), jnp.float32),
        scratch_shapes=[pltpu.SMEM((4,), jnp.float32)],
    )(x, t, rg, rt)
    return out[0, 0]
